# baseline (speedup 1.0000x reference)
import jax
import jax.numpy as jnp
from jax.experimental import pallas as pl
from jax.experimental.pallas import tpu as pltpu

N = 8192
ROWS = 4096
BLK = 2048
HI = 128
LO = 64
WEIGHT = 0.999


def _hist_kernel(ml_ref, mr_ref, p_ref, out_ref):
    i = pl.program_id(0)
    ones = jnp.ones((N // 2, 1), dtype=jnp.int8)
    lens_l = jax.lax.dot_general(
        ml_ref[...], ones, (((1,), (0,)), ((), ())),
        preferred_element_type=jnp.int32)
    lens_r = jax.lax.dot_general(
        mr_ref[...], ones, (((1,), (0,)), ((), ())),
        preferred_element_type=jnp.int32)
    t = lens_l + lens_r - 1
    hi = t >> 6
    lo = t & (LO - 1)
    hiota = jax.lax.broadcasted_iota(jnp.int32, (1, HI), 1)
    loiota = jax.lax.broadcasted_iota(jnp.int32, (1, LO), 1)
    u = (hi == hiota).astype(jnp.bfloat16)
    v = (lo == loiota).astype(jnp.bfloat16)
    part = jax.lax.dot_general(
        u, v, (((0,), (0,)), ((), ())),
        preferred_element_type=jnp.float32)

    @pl.when(i == 0)
    def _init():
        out_ref[...] = jnp.zeros_like(out_ref)

    out_ref[...] += part

    @pl.when(i == pl.num_programs(0) - 1)
    def _finish():
        out_ref[...] = WEIGHT * p_ref[...] + ((1.0 - WEIGHT) / ROWS) * out_ref[...]


def kernel(mask, n_elements_prob):
    m8 = mask.view(jnp.int8)
    p2 = n_elements_prob.reshape(HI, LO)
    out = pl.pallas_call(
        _hist_kernel,
        grid=(ROWS // BLK,),
        in_specs=[
            pl.BlockSpec((BLK, N // 2), lambda i: (i, 0)),
            pl.BlockSpec((BLK, N // 2), lambda i: (i, 1)),
            pl.BlockSpec((HI, LO), lambda i: (0, 0)),
        ],
        out_specs=pl.BlockSpec((HI, LO), lambda i: (0, 0)),
        compiler_params=pltpu.CompilerParams(
            dimension_semantics=("arbitrary",)),
        out_shape=jax.ShapeDtypeStruct((HI, LO), jnp.float32),
    )(m8, m8, p2)
    return out.reshape(N)


# speedup vs baseline: 1.0338x; 1.0338x over previous
import jax
import jax.numpy as jnp
from jax.experimental import pallas as pl
from jax.experimental.pallas import tpu as pltpu

N = 8192
ROWS = 4096
BLK = 1024
HI = 128
LO = 64
WEIGHT = 0.999


def _hist_kernel(ml_ref, mr_ref, p_ref, out_ref):
    i = pl.program_id(0)
    ones = jnp.ones((N // 2, 1), dtype=jnp.int8)
    lens_l = jax.lax.dot_general(
        ml_ref[...], ones, (((1,), (0,)), ((), ())),
        preferred_element_type=jnp.int32)
    lens_r = jax.lax.dot_general(
        mr_ref[...], ones, (((1,), (0,)), ((), ())),
        preferred_element_type=jnp.int32)
    t = lens_l + lens_r - 1
    hi = t >> 6
    lo = t & (LO - 1)
    hiota = jax.lax.broadcasted_iota(jnp.int32, (1, HI), 1)
    loiota = jax.lax.broadcasted_iota(jnp.int32, (1, LO), 1)
    u = (hi == hiota).astype(jnp.bfloat16)
    v = (lo == loiota).astype(jnp.bfloat16)
    part = jax.lax.dot_general(
        u, v, (((0,), (0,)), ((), ())),
        preferred_element_type=jnp.float32)

    @pl.when(i == 0)
    def _init():
        out_ref[...] = jnp.zeros_like(out_ref)

    out_ref[...] += part

    @pl.when(i == pl.num_programs(0) - 1)
    def _finish():
        out_ref[...] = WEIGHT * p_ref[...] + ((1.0 - WEIGHT) / ROWS) * out_ref[...]


def kernel(mask, n_elements_prob):
    m8 = mask.view(jnp.int8)
    p2 = n_elements_prob.reshape(HI, LO)
    out = pl.pallas_call(
        _hist_kernel,
        grid=(ROWS // BLK,),
        in_specs=[
            pl.BlockSpec((BLK, N // 2), lambda i: (i, 0)),
            pl.BlockSpec((BLK, N // 2), lambda i: (i, 1)),
            pl.BlockSpec((HI, LO), lambda i: (0, 0)),
        ],
        out_specs=pl.BlockSpec((HI, LO), lambda i: (0, 0)),
        compiler_params=pltpu.CompilerParams(
            dimension_semantics=("arbitrary",)),
        out_shape=jax.ShapeDtypeStruct((HI, LO), jnp.float32),
    )(m8, m8, p2)
    return out.reshape(N)
